# baseline (speedup 1.0000x reference)
import functools

import jax
import jax.numpy as jnp
from jax import lax
from jax.experimental import pallas as pl
from jax.experimental.pallas import tpu as pltpu
from jax.experimental.pallas import tpu_sc as plsc

N = 10000
D_IN = 128
NP = 10240
NC = 2
NS = 16
NW = NC * NS
E1 = 320000 + N
EP = 331776
RPT = NP // NS


@functools.cache
def _sc_edge_kernel(c: int):
    mesh = plsc.VectorSubcoreMesh(
        core_axis_name="c", subcore_axis_name="s", num_cores=NC, num_subcores=NS
    )
    blk = 128 if c <= 64 else 64
    bpw = EP // (NW * blk)

    def body(h_hbm, as_hbm, ad_hbm, src_hbm, dst_hbm,
             u_out, den_out,
             src_t, dst_t, as_t, ad_t, ex_t, rows_t, zden_t,
             u_sh, den_sh, sem_g, sem_s, sem_d, sem_u, sem_e):
        cid = lax.axis_index("c")
        sid = lax.axis_index("s")
        wid = cid * NS + sid

        def idx_fetch(b, slot):
            pltpu.async_copy(src_hbm.at[wid, b], src_t.at[slot],
                             sem_s.at[slot])
            pltpu.async_copy(dst_hbm.at[wid, b], dst_t.at[slot],
                             sem_d.at[slot])

        def idx_wait(slot):
            pltpu.make_async_copy(src_hbm.at[wid, 0], src_t.at[slot],
                                  sem_s.at[slot]).wait()
            pltpu.make_async_copy(dst_hbm.at[wid, 0], dst_t.at[slot],
                                  sem_d.at[slot]).wait()

        def gather_start(b, buf):
            pltpu.async_copy(h_hbm.at[src_t.at[lax.rem(b, 4)]],
                             rows_t.at[buf], sem_g)

        def gather_wait(buf):
            pltpu.make_async_copy(h_hbm.at[src_t.at[0]], rows_t.at[buf],
                                  sem_g).wait()

        def scatter_start(slot, buf):
            pltpu.async_copy(rows_t.at[buf], u_sh.at[dst_t.at[slot]],
                             sem_u.at[buf], add=True)
            pltpu.async_copy(ex_t.at[buf], den_sh.at[dst_t.at[slot]],
                             sem_e.at[buf], add=True)

        def scatter_wait(buf):
            pltpu.make_async_copy(rows_t.at[buf], u_sh.at[dst_t.at[0]],
                                  sem_u.at[buf]).wait()
            pltpu.make_async_copy(ex_t.at[buf], den_sh.at[dst_t.at[0]],
                                  sem_e.at[buf]).wait()

        idx_fetch(0, 0)
        idx_fetch(1, 1)
        pltpu.sync_copy(as_hbm, as_t)
        pltpu.sync_copy(ad_hbm, ad_t)

        zero = jnp.zeros((16,), jnp.float32)

        def zrow(r, _):
            for j in range(c // 16):
                rows_t[0, r, pl.ds(j * 16, 16)] = zero
            return 0

        lax.fori_loop(0, blk, zrow, 0)

        def zden(i, _):
            zden_t[pl.ds(i * 16, 16)] = zero
            return 0

        lax.fori_loop(0, RPT // 16, zden, 0)

        for i in range(RPT // blk):
            pltpu.sync_copy(rows_t.at[0],
                            u_sh.at[pl.ds(sid * RPT + i * blk, blk)])
        pltpu.sync_copy(zden_t, den_sh.at[pl.ds(sid * RPT, RPT)])
        plsc.subcore_barrier()

        idx_wait(0)
        gather_start(0, 0)

        def blk_body(b, _):
            slot = lax.rem(b, 4)
            buf = lax.rem(b, 2)

            def g_body(g, _):
                s = src_t[slot, pl.ds(g * 16, 16)]
                d = dst_t[slot, pl.ds(g * 16, 16)]
                al = plsc.load_gather(as_t, [s]) + plsc.load_gather(ad_t, [d])
                al = jnp.where(al >= 0, al, al * jnp.float32(0.2))
                ex_t[buf, pl.ds(g * 16, 16)] = jnp.exp(al)
                return 0

            lax.fori_loop(0, blk // 16, g_body, 0)

            gather_wait(buf)

            def s_body(g, _):
                exv = ex_t[buf, pl.ds(g * 16, 16)]
                for r in range(16):
                    es = exv[jnp.full((16,), r, jnp.int32)]
                    row = g * 16 + r
                    for j in range(c // 16):
                        rows_t[buf, row, pl.ds(j * 16, 16)] = (
                            rows_t[buf, row, pl.ds(j * 16, 16)] * es
                        )
                return 0

            lax.fori_loop(0, blk // 16, s_body, 0)

            @pl.when(b >= 1)
            def _drain_prev():
                scatter_wait(1 - buf)

            @pl.when(b + 1 < bpw)
            def _next_gather():
                idx_wait(lax.rem(b + 1, 4))
                gather_start(b + 1, 1 - buf)

            @pl.when(b + 2 < bpw)
            def _next_idx():
                idx_fetch(b + 2, lax.rem(b + 2, 4))

            scatter_start(slot, buf)
            return 0

        lax.fori_loop(0, bpw, blk_body, 0)
        scatter_wait(lax.rem(bpw - 1, 2))
        plsc.subcore_barrier()

        pltpu.sync_copy(u_sh.at[pl.ds(sid * RPT, RPT)],
                        u_out.at[cid, pl.ds(sid * RPT, RPT)])
        pltpu.sync_copy(den_sh.at[pl.ds(sid * RPT, RPT)],
                        den_out.at[cid, pl.ds(sid * RPT, RPT)])

    return pl.kernel(
        body,
        out_type=(
            jax.ShapeDtypeStruct((NC, NP, c), jnp.float32),
            jax.ShapeDtypeStruct((NC, NP), jnp.float32),
        ),
        mesh=mesh,
        compiler_params=pltpu.CompilerParams(
            needs_layout_passes=False, use_tc_tiling_on_sc=False),
        scratch_types=[
            pltpu.VMEM((4, blk), jnp.int32),
            pltpu.VMEM((4, blk), jnp.int32),
            pltpu.VMEM((NP,), jnp.float32),
            pltpu.VMEM((NP,), jnp.float32),
            pltpu.VMEM((2, blk), jnp.float32),
            pltpu.VMEM((2, blk, c), jnp.float32),
            pltpu.VMEM((RPT,), jnp.float32),
            pltpu.VMEM_SHARED((NP, c), jnp.float32),
            pltpu.VMEM_SHARED((NP,), jnp.float32),
            pltpu.SemaphoreType.DMA,
            pltpu.SemaphoreType.DMA((4,)),
            pltpu.SemaphoreType.DMA((4,)),
            pltpu.SemaphoreType.DMA((2,)),
            pltpu.SemaphoreType.DMA((2,)),
        ],
    )


_BN = 1024


def _tc0_body(x_ref, w_ref, av_ref, bv_ref, h_ref, s_ref, d_ref):
    h = jnp.dot(x_ref[...], w_ref[...], preferred_element_type=jnp.float32)
    h_ref[...] = h
    s_ref[...] = jnp.dot(h, av_ref[...], preferred_element_type=jnp.float32)
    d_ref[...] = jnp.dot(h, bv_ref[...], preferred_element_type=jnp.float32)


def _tc_mid_body(u0_ref, u1_ref, n0_ref, n1_ref, b_ref, w_ref, av_ref, bv_ref,
                 h_ref, s_ref, d_ref):
    den = n0_ref[...] + n1_ref[...] + jnp.float32(1e-16)
    xact = jnp.maximum((u0_ref[...] + u1_ref[...]) / den + b_ref[...], 0.0)
    h = jnp.dot(xact, w_ref[...], preferred_element_type=jnp.float32)
    h_ref[...] = h
    s_ref[...] = jnp.dot(h, av_ref[...], preferred_element_type=jnp.float32)
    d_ref[...] = jnp.dot(h, bv_ref[...], preferred_element_type=jnp.float32)


def _tc_fin_body(u0_ref, u1_ref, n0_ref, n1_ref, b_ref, o_ref):
    den = n0_ref[...] + n1_ref[...] + jnp.float32(1e-16)
    o_ref[...] = (u0_ref[...] + u1_ref[...]) / den + b_ref[...]


def _row_spec(c):
    return pl.BlockSpec((_BN, c), lambda i: (i, 0))


def _full_spec(shape):
    return pl.BlockSpec(shape, lambda i: tuple(0 for _ in shape))


def _tc0(x, w, av, bv):
    cin, cout = w.shape
    return pl.pallas_call(
        _tc0_body,
        grid=(NP // _BN,),
        in_specs=[_row_spec(cin), _full_spec(w.shape), _full_spec(av.shape),
                  _full_spec(bv.shape)],
        out_specs=[_row_spec(cout), _row_spec(1), _row_spec(1)],
        out_shape=[
            jax.ShapeDtypeStruct((NP, cout), jnp.float32),
            jax.ShapeDtypeStruct((NP, 1), jnp.float32),
            jax.ShapeDtypeStruct((NP, 1), jnp.float32),
        ],
    )(x, w, av, bv)


def _tc_mid(u, den, b, w, av, bv):
    cin, cout = w.shape
    return pl.pallas_call(
        _tc_mid_body,
        grid=(NP // _BN,),
        in_specs=[_row_spec(cin), _row_spec(cin), _row_spec(1), _row_spec(1),
                  _full_spec((1, cin)), _full_spec(w.shape),
                  _full_spec(av.shape), _full_spec(bv.shape)],
        out_specs=[_row_spec(cout), _row_spec(1), _row_spec(1)],
        out_shape=[
            jax.ShapeDtypeStruct((NP, cout), jnp.float32),
            jax.ShapeDtypeStruct((NP, 1), jnp.float32),
            jax.ShapeDtypeStruct((NP, 1), jnp.float32),
        ],
    )(u[0], u[1], den[0].reshape(NP, 1), den[1].reshape(NP, 1),
      b.reshape(1, cin), w, av, bv)


def _tc_fin(u, den, b):
    cin = u.shape[-1]
    return pl.pallas_call(
        _tc_fin_body,
        grid=(NP // _BN,),
        in_specs=[_row_spec(cin), _row_spec(cin), _row_spec(1), _row_spec(1),
                  _full_spec((1, cin))],
        out_specs=_row_spec(cin),
        out_shape=jax.ShapeDtypeStruct((NP, cin), jnp.float32),
    )(u[0], u[1], den[0].reshape(NP, 1), den[1].reshape(NP, 1),
      b.reshape(1, cin))


def kernel(x, edge_index, W0, a_src0, a_dst0, b0, W1, a_src1, a_dst1, b1,
           W2, a_src2, a_dst2, b2):
    loop = jnp.arange(N, dtype=edge_index.dtype)
    src = jnp.concatenate([edge_index[0], loop])
    dst = jnp.concatenate([edge_index[1], loop])
    pad = jnp.full((EP - E1,), N, dtype=edge_index.dtype)
    src_f = jnp.concatenate([src, pad])
    dst_f = jnp.concatenate([dst, pad])

    xp = jnp.pad(x, ((0, NP - N), (0, 0)))

    def sc_call(h, s, d):
        c = h.shape[-1]
        blk = 128 if c <= 64 else 64
        e3 = (NW, EP // (NW * blk), blk)
        return _sc_edge_kernel(c)(h, s.reshape(NP), d.reshape(NP),
                                  src_f.reshape(e3), dst_f.reshape(e3))

    h, s, d = _tc0(xp, W0, a_src0.reshape(-1, 1), a_dst0.reshape(-1, 1))
    u, den = sc_call(h, s, d)
    h, s, d = _tc_mid(u, den, b0, W1, a_src1.reshape(-1, 1),
                      a_dst1.reshape(-1, 1))
    u, den = sc_call(h, s, d)
    h, s, d = _tc_mid(u, den, b1, W2, a_src2.reshape(-1, 1),
                      a_dst2.reshape(-1, 1))
    u, den = sc_call(h, s, d)
    out = _tc_fin(u, den, b2)
    return out[:N]


# speedup vs baseline: 1.4870x; 1.4870x over previous
import functools

import jax
import jax.numpy as jnp
from jax import lax
from jax.experimental import pallas as pl
from jax.experimental.pallas import tpu as pltpu
from jax.experimental.pallas import tpu_sc as plsc

N = 10000
D_IN = 128
NP = 10240
NC = 2
NS = 16
NW = NC * NS
E1 = 320000 + N
EP = 331776
RPT = NP // NS


@functools.cache
def _sc_edge_kernel(c: int):
    mesh = plsc.VectorSubcoreMesh(
        core_axis_name="c", subcore_axis_name="s", num_cores=NC, num_subcores=NS
    )
    blk = 128
    bpw = EP // (NW * blk)
    packed = c > 64

    def body(h_hbm, as_hbm, ad_hbm, src_hbm, dst_hbm,
             u_out, den_out,
             src_t, dst_t, as_t, ad_t, ex_t, rows_t, zden_t,
             u_sh, den_sh, sem_g, sem_s, sem_d, sem_u, sem_e):
        cid = lax.axis_index("c")
        sid = lax.axis_index("s")
        wid = cid * NS + sid

        def idx_fetch(b, slot):
            pltpu.async_copy(src_hbm.at[wid, b], src_t.at[slot],
                             sem_s.at[slot])
            pltpu.async_copy(dst_hbm.at[wid, b], dst_t.at[slot],
                             sem_d.at[slot])

        def idx_wait(slot):
            pltpu.make_async_copy(src_hbm.at[wid, 0], src_t.at[slot],
                                  sem_s.at[slot]).wait()
            pltpu.make_async_copy(dst_hbm.at[wid, 0], dst_t.at[slot],
                                  sem_d.at[slot]).wait()

        def gather_start(b, buf):
            pltpu.async_copy(h_hbm.at[src_t.at[lax.rem(b, 4)]],
                             rows_t.at[buf], sem_g)

        def gather_wait(buf):
            pltpu.make_async_copy(h_hbm.at[src_t.at[0]], rows_t.at[buf],
                                  sem_g).wait()

        def scatter_start(slot, buf):
            pltpu.async_copy(rows_t.at[buf], u_sh.at[dst_t.at[slot]],
                             sem_u.at[buf], add=True)
            pltpu.async_copy(ex_t.at[buf], den_sh.at[dst_t.at[slot]],
                             sem_e.at[buf], add=True)

        def scatter_wait(buf):
            pltpu.make_async_copy(rows_t.at[buf], u_sh.at[dst_t.at[0]],
                                  sem_u.at[buf]).wait()
            pltpu.make_async_copy(ex_t.at[buf], den_sh.at[dst_t.at[0]],
                                  sem_e.at[buf]).wait()

        idx_fetch(0, 0)
        idx_fetch(1, 1)
        pltpu.sync_copy(as_hbm, as_t)
        if not packed:
            pltpu.sync_copy(ad_hbm, ad_t)

        zero = jnp.zeros((16,), jnp.float32)

        @plsc.parallel_loop(0, blk)
        def _zrow(r):
            for j in range(c // 16):
                rows_t[0, r, pl.ds(j * 16, 16)] = zero

        @plsc.parallel_loop(0, RPT // 16)
        def _zden(i):
            zden_t[pl.ds(i * 16, 16)] = zero

        full, rem = divmod(RPT, blk)
        for i in range(full):
            pltpu.sync_copy(rows_t.at[0],
                            u_sh.at[pl.ds(sid * RPT + i * blk, blk)])
        if rem:
            pltpu.sync_copy(rows_t.at[0, pl.ds(0, rem)],
                            u_sh.at[pl.ds(sid * RPT + full * blk, rem)])
        pltpu.sync_copy(zden_t, den_sh.at[pl.ds(sid * RPT, RPT)])
        plsc.subcore_barrier()

        idx_wait(0)
        gather_start(0, 0)

        def blk_body(b, _):
            slot = lax.rem(b, 4)
            buf = lax.rem(b, 2)

            @plsc.parallel_loop(0, blk // 16, unroll=2)
            def _alpha(g):
                s = src_t[slot, pl.ds(g * 16, 16)]
                d = dst_t[slot, pl.ds(g * 16, 16)]
                if packed:
                    ws = plsc.load_gather(as_t, [s])
                    wd = plsc.load_gather(as_t, [d])
                    av = lax.bitcast_convert_type(
                        ws & jnp.int32(-65536), jnp.float32)
                    dv = lax.bitcast_convert_type(
                        lax.shift_left(wd, 16), jnp.float32)
                    al = av + dv
                else:
                    al = (plsc.load_gather(as_t, [s])
                          + plsc.load_gather(ad_t, [d]))
                al = jnp.where(al >= 0, al, al * jnp.float32(0.2))
                ex_t[buf, pl.ds(g * 16, 16)] = jnp.exp(al)

            gather_wait(buf)

            @plsc.parallel_loop(0, blk // 16, unroll=2)
            def _scale(g):
                exv = ex_t[buf, pl.ds(g * 16, 16)]
                for r in range(16):
                    es = exv[jnp.full((16,), r, jnp.int32)]
                    row = g * 16 + r
                    for j in range(c // 16):
                        rows_t[buf, row, pl.ds(j * 16, 16)] = (
                            rows_t[buf, row, pl.ds(j * 16, 16)] * es
                        )

            @pl.when(b >= 1)
            def _drain_prev():
                scatter_wait(1 - buf)

            @pl.when(b + 1 < bpw)
            def _next_gather():
                idx_wait(lax.rem(b + 1, 4))
                gather_start(b + 1, 1 - buf)

            @pl.when(b + 2 < bpw)
            def _next_idx():
                idx_fetch(b + 2, lax.rem(b + 2, 4))

            scatter_start(slot, buf)
            return 0

        lax.fori_loop(0, bpw, blk_body, 0)
        scatter_wait(lax.rem(bpw - 1, 2))
        plsc.subcore_barrier()

        pltpu.sync_copy(u_sh.at[pl.ds(sid * RPT, RPT)],
                        u_out.at[cid, pl.ds(sid * RPT, RPT)])
        pltpu.sync_copy(den_sh.at[pl.ds(sid * RPT, RPT)],
                        den_out.at[cid, pl.ds(sid * RPT, RPT)])

    return pl.kernel(
        body,
        out_type=(
            jax.ShapeDtypeStruct((NC, NP, c), jnp.float32),
            jax.ShapeDtypeStruct((NC, NP), jnp.float32),
        ),
        mesh=mesh,
        compiler_params=pltpu.CompilerParams(
            needs_layout_passes=False, use_tc_tiling_on_sc=False),
        scratch_types=[
            pltpu.VMEM((4, blk), jnp.int32),
            pltpu.VMEM((4, blk), jnp.int32),
            pltpu.VMEM((NP,), jnp.int32 if packed else jnp.float32),
            pltpu.VMEM((16,) if packed else (NP,), jnp.float32),
            pltpu.VMEM((2, blk), jnp.float32),
            pltpu.VMEM((2, blk, c), jnp.float32),
            pltpu.VMEM((RPT,), jnp.float32),
            pltpu.VMEM_SHARED((NP, c), jnp.float32),
            pltpu.VMEM_SHARED((NP,), jnp.float32),
            pltpu.SemaphoreType.DMA,
            pltpu.SemaphoreType.DMA((4,)),
            pltpu.SemaphoreType.DMA((4,)),
            pltpu.SemaphoreType.DMA((2,)),
            pltpu.SemaphoreType.DMA((2,)),
        ],
    )


_BN = 1024


def _tc0_body(x_ref, w_ref, av_ref, bv_ref, h_ref, s_ref, d_ref):
    h = jnp.dot(x_ref[...], w_ref[...], preferred_element_type=jnp.float32)
    h_ref[...] = h
    s_ref[...] = jnp.dot(h, av_ref[...], preferred_element_type=jnp.float32)
    d_ref[...] = jnp.dot(h, bv_ref[...], preferred_element_type=jnp.float32)


def _tc_mid_body(u0_ref, u1_ref, n0_ref, n1_ref, b_ref, w_ref, av_ref, bv_ref,
                 h_ref, s_ref, d_ref):
    den = n0_ref[...] + n1_ref[...] + jnp.float32(1e-16)
    xact = jnp.maximum((u0_ref[...] + u1_ref[...]) / den + b_ref[...], 0.0)
    h = jnp.dot(xact, w_ref[...], preferred_element_type=jnp.float32)
    h_ref[...] = h
    s_ref[...] = jnp.dot(h, av_ref[...], preferred_element_type=jnp.float32)
    d_ref[...] = jnp.dot(h, bv_ref[...], preferred_element_type=jnp.float32)


def _tc_fin_body(u0_ref, u1_ref, n0_ref, n1_ref, b_ref, o_ref):
    den = n0_ref[...] + n1_ref[...] + jnp.float32(1e-16)
    o_ref[...] = (u0_ref[...] + u1_ref[...]) / den + b_ref[...]


def _row_spec(c):
    return pl.BlockSpec((_BN, c), lambda i: (i, 0))


def _full_spec(shape):
    return pl.BlockSpec(shape, lambda i: tuple(0 for _ in shape))


def _tc0(x, w, av, bv):
    cin, cout = w.shape
    return pl.pallas_call(
        _tc0_body,
        grid=(NP // _BN,),
        in_specs=[_row_spec(cin), _full_spec(w.shape), _full_spec(av.shape),
                  _full_spec(bv.shape)],
        out_specs=[_row_spec(cout), _row_spec(1), _row_spec(1)],
        out_shape=[
            jax.ShapeDtypeStruct((NP, cout), jnp.float32),
            jax.ShapeDtypeStruct((NP, 1), jnp.float32),
            jax.ShapeDtypeStruct((NP, 1), jnp.float32),
        ],
    )(x, w, av, bv)


def _tc_mid(u, den, b, w, av, bv):
    cin, cout = w.shape
    return pl.pallas_call(
        _tc_mid_body,
        grid=(NP // _BN,),
        in_specs=[_row_spec(cin), _row_spec(cin), _row_spec(1), _row_spec(1),
                  _full_spec((1, cin)), _full_spec(w.shape),
                  _full_spec(av.shape), _full_spec(bv.shape)],
        out_specs=[_row_spec(cout), _row_spec(1), _row_spec(1)],
        out_shape=[
            jax.ShapeDtypeStruct((NP, cout), jnp.float32),
            jax.ShapeDtypeStruct((NP, 1), jnp.float32),
            jax.ShapeDtypeStruct((NP, 1), jnp.float32),
        ],
    )(u[0], u[1], den[0].reshape(NP, 1), den[1].reshape(NP, 1),
      b.reshape(1, cin), w, av, bv)


def _tc_fin(u, den, b):
    cin = u.shape[-1]
    return pl.pallas_call(
        _tc_fin_body,
        grid=(NP // _BN,),
        in_specs=[_row_spec(cin), _row_spec(cin), _row_spec(1), _row_spec(1),
                  _full_spec((1, cin))],
        out_specs=_row_spec(cin),
        out_shape=jax.ShapeDtypeStruct((NP, cin), jnp.float32),
    )(u[0], u[1], den[0].reshape(NP, 1), den[1].reshape(NP, 1),
      b.reshape(1, cin))


def kernel(x, edge_index, W0, a_src0, a_dst0, b0, W1, a_src1, a_dst1, b1,
           W2, a_src2, a_dst2, b2):
    loop = jnp.arange(N, dtype=edge_index.dtype)
    src = jnp.concatenate([edge_index[0], loop])
    dst = jnp.concatenate([edge_index[1], loop])
    pad = jnp.full((EP - E1,), N, dtype=edge_index.dtype)
    src_f = jnp.concatenate([src, pad])
    dst_f = jnp.concatenate([dst, pad])

    xp = jnp.pad(x, ((0, NP - N), (0, 0)))

    def sc_call(h, s, d):
        c = h.shape[-1]
        e3 = (NW, EP // (NW * 128), 128)
        sv, dv = s.reshape(NP), d.reshape(NP)
        if c > 64:
            su = lax.bitcast_convert_type(
                sv.astype(jnp.bfloat16), jnp.uint16).astype(jnp.uint32)
            du = lax.bitcast_convert_type(
                dv.astype(jnp.bfloat16), jnp.uint16).astype(jnp.uint32)
            aa = lax.bitcast_convert_type((su << 16) | du, jnp.int32)
            sv = dv = aa
        return _sc_edge_kernel(c)(h, sv, dv,
                                  src_f.reshape(e3), dst_f.reshape(e3))

    h, s, d = _tc0(xp, W0, a_src0.reshape(-1, 1), a_dst0.reshape(-1, 1))
    u, den = sc_call(h, s, d)
    h, s, d = _tc_mid(u, den, b0, W1, a_src1.reshape(-1, 1),
                      a_dst1.reshape(-1, 1))
    u, den = sc_call(h, s, d)
    h, s, d = _tc_mid(u, den, b1, W2, a_src2.reshape(-1, 1),
                      a_dst2.reshape(-1, 1))
    u, den = sc_call(h, s, d)
    out = _tc_fin(u, den, b2)
    return out[:N]


# speedup vs baseline: 1.8731x; 1.2597x over previous
import functools

import jax
import jax.numpy as jnp
from jax import lax
from jax.experimental import pallas as pl
from jax.experimental.pallas import tpu as pltpu
from jax.experimental.pallas import tpu_sc as plsc

N = 10000
D_IN = 128
NP = 10240
NC = 2
NS = 16
NW = NC * NS
E1 = 320000 + N
EP = 331776
RPT = NP // NS


@functools.cache
def _sc_edge_kernel(c: int):
    mesh = plsc.VectorSubcoreMesh(
        core_axis_name="c", subcore_axis_name="s", num_cores=NC, num_subcores=NS
    )
    blk = 128
    bpw = EP // (NW * blk)
    packed = c > 64

    def body(h_hbm, as_hbm, ad_hbm, src_hbm, dst_hbm,
             u_out, den_out,
             src_t, dst_t, as_t, ad_t, ex_t, rows_t, zden_t,
             u_sh, den_sh, sem_g, sem_s, sem_d, sem_u, sem_e):
        cid = lax.axis_index("c")
        sid = lax.axis_index("s")
        wid = cid * NS + sid

        def idx_fetch(b, slot):
            pltpu.async_copy(src_hbm.at[wid, b], src_t.at[slot],
                             sem_s.at[slot])
            pltpu.async_copy(dst_hbm.at[wid, b], dst_t.at[slot],
                             sem_d.at[slot])

        def idx_wait(slot):
            pltpu.make_async_copy(src_hbm.at[wid, 0], src_t.at[slot],
                                  sem_s.at[slot]).wait()
            pltpu.make_async_copy(dst_hbm.at[wid, 0], dst_t.at[slot],
                                  sem_d.at[slot]).wait()

        def gather_start(b, buf):
            pltpu.async_copy(h_hbm.at[src_t.at[lax.rem(b, 4)]],
                             rows_t.at[buf], sem_g)

        def gather_wait(buf):
            pltpu.make_async_copy(h_hbm.at[src_t.at[0]], rows_t.at[buf],
                                  sem_g).wait()

        def scatter_start(slot, buf):
            pltpu.async_copy(rows_t.at[buf], u_sh.at[dst_t.at[slot]],
                             sem_u.at[buf], add=True)
            pltpu.async_copy(ex_t.at[buf], den_sh.at[dst_t.at[slot]],
                             sem_e.at[buf], add=True)

        def scatter_wait(buf):
            pltpu.make_async_copy(rows_t.at[buf], u_sh.at[dst_t.at[0]],
                                  sem_u.at[buf]).wait()
            pltpu.make_async_copy(ex_t.at[buf], den_sh.at[dst_t.at[0]],
                                  sem_e.at[buf]).wait()

        idx_fetch(0, 0)
        idx_fetch(1, 1)
        pltpu.sync_copy(as_hbm, as_t)
        if not packed:
            pltpu.sync_copy(ad_hbm, ad_t)

        zero = jnp.zeros((16,), jnp.float32)

        @plsc.parallel_loop(0, blk)
        def _zrow(r):
            for j in range(c // 16):
                rows_t[0, r, pl.ds(j * 16, 16)] = zero

        @plsc.parallel_loop(0, RPT // 16)
        def _zden(i):
            zden_t[pl.ds(i * 16, 16)] = zero

        full, rem = divmod(RPT, blk)
        for i in range(full):
            pltpu.sync_copy(rows_t.at[0],
                            u_sh.at[pl.ds(sid * RPT + i * blk, blk)])
        if rem:
            pltpu.sync_copy(rows_t.at[0, pl.ds(0, rem)],
                            u_sh.at[pl.ds(sid * RPT + full * blk, rem)])
        pltpu.sync_copy(zden_t, den_sh.at[pl.ds(sid * RPT, RPT)])
        plsc.subcore_barrier()

        idx_wait(0)
        gather_start(0, 0)

        def blk_body(b, _):
            slot = lax.rem(b, 4)
            buf = lax.rem(b, 2)

            @plsc.parallel_loop(0, blk // 16, unroll=2)
            def _alpha(g):
                s = src_t[slot, pl.ds(g * 16, 16)]
                d = dst_t[slot, pl.ds(g * 16, 16)]
                if packed:
                    ws = plsc.load_gather(as_t, [s])
                    wd = plsc.load_gather(as_t, [d])
                    av = lax.bitcast_convert_type(
                        ws & jnp.int32(-65536), jnp.float32)
                    dv = lax.bitcast_convert_type(
                        lax.shift_left(wd, 16), jnp.float32)
                    al = av + dv
                else:
                    al = (plsc.load_gather(as_t, [s])
                          + plsc.load_gather(ad_t, [d]))
                al = jnp.where(al >= 0, al, al * jnp.float32(0.2))
                ex_t[buf, pl.ds(g * 16, 16)] = jnp.exp(al)

            gather_wait(buf)

            @plsc.parallel_loop(0, blk // 16, unroll=2)
            def _scale(g):
                exv = ex_t[buf, pl.ds(g * 16, 16)]
                for r in range(16):
                    es = exv[jnp.full((16,), r, jnp.int32)]
                    row = g * 16 + r
                    for j in range(c // 16):
                        rows_t[buf, row, pl.ds(j * 16, 16)] = (
                            rows_t[buf, row, pl.ds(j * 16, 16)] * es
                        )

            @pl.when(b >= 1)
            def _drain_prev():
                scatter_wait(1 - buf)

            @pl.when(b + 1 < bpw)
            def _next_gather():
                idx_wait(lax.rem(b + 1, 4))
                gather_start(b + 1, 1 - buf)

            @pl.when(b + 2 < bpw)
            def _next_idx():
                idx_fetch(b + 2, lax.rem(b + 2, 4))

            scatter_start(slot, buf)
            return 0

        lax.fori_loop(0, bpw, blk_body, 0)
        scatter_wait(lax.rem(bpw - 1, 2))
        plsc.subcore_barrier()

        pltpu.sync_copy(u_sh.at[pl.ds(sid * RPT, RPT)],
                        u_out.at[cid, pl.ds(sid * RPT, RPT)])
        pltpu.sync_copy(den_sh.at[pl.ds(sid * RPT, RPT)],
                        den_out.at[cid, pl.ds(sid * RPT, RPT)])

    return pl.kernel(
        body,
        out_type=(
            jax.ShapeDtypeStruct((NC, NP, c), jnp.float32),
            jax.ShapeDtypeStruct((NC, NP), jnp.float32),
        ),
        mesh=mesh,
        compiler_params=pltpu.CompilerParams(
            needs_layout_passes=False, use_tc_tiling_on_sc=False),
        scratch_types=[
            pltpu.VMEM((4, blk), jnp.int32),
            pltpu.VMEM((4, blk), jnp.int32),
            pltpu.VMEM((NP,), jnp.int32 if packed else jnp.float32),
            pltpu.VMEM((16,) if packed else (NP,), jnp.float32),
            pltpu.VMEM((2, blk), jnp.float32),
            pltpu.VMEM((2, blk, c), jnp.float32),
            pltpu.VMEM((RPT,), jnp.float32),
            pltpu.VMEM_SHARED((NP, c), jnp.float32),
            pltpu.VMEM_SHARED((NP,), jnp.float32),
            pltpu.SemaphoreType.DMA,
            pltpu.SemaphoreType.DMA((4,)),
            pltpu.SemaphoreType.DMA((4,)),
            pltpu.SemaphoreType.DMA((2,)),
            pltpu.SemaphoreType.DMA((2,)),
        ],
    )


_BN = 1024


def _tc0_body(x_ref, w_ref, av_ref, bv_ref, h_ref, s_ref, d_ref):
    h = jnp.dot(x_ref[...], w_ref[...], preferred_element_type=jnp.float32)
    h_ref[...] = h
    s_ref[...] = jnp.dot(h, av_ref[...], preferred_element_type=jnp.float32)
    d_ref[...] = jnp.dot(h, bv_ref[...], preferred_element_type=jnp.float32)


def _tc_mid_body(u0_ref, u1_ref, n0_ref, n1_ref, b_ref, w_ref, av_ref, bv_ref,
                 h_ref, s_ref, d_ref):
    den = n0_ref[...] + n1_ref[...] + jnp.float32(1e-16)
    xact = jnp.maximum((u0_ref[...] + u1_ref[...]) / den + b_ref[...], 0.0)
    h = jnp.dot(xact, w_ref[...], preferred_element_type=jnp.float32)
    h_ref[...] = h
    s_ref[...] = jnp.dot(h, av_ref[...], preferred_element_type=jnp.float32)
    d_ref[...] = jnp.dot(h, bv_ref[...], preferred_element_type=jnp.float32)


def _tc_pre_body(u0_ref, u1_ref, n0_ref, n1_ref, b_ref, w_ref, av_ref, bv_ref,
                 h_ref, s_ref, d_ref):
    den = n0_ref[...] + n1_ref[...] + jnp.float32(1e-16)
    xact = jnp.maximum((u0_ref[...] + u1_ref[...]) / den + b_ref[...], 0.0)
    h_ref[...] = xact
    wa = jnp.dot(w_ref[...], av_ref[...], preferred_element_type=jnp.float32)
    wb = jnp.dot(w_ref[...], bv_ref[...], preferred_element_type=jnp.float32)
    s_ref[...] = jnp.dot(xact, wa, preferred_element_type=jnp.float32)
    d_ref[...] = jnp.dot(xact, wb, preferred_element_type=jnp.float32)


def _tc_fin_body(u0_ref, u1_ref, n0_ref, n1_ref, b_ref, w_ref, o_ref):
    den = n0_ref[...] + n1_ref[...] + jnp.float32(1e-16)
    agg = jnp.dot(u0_ref[...] + u1_ref[...], w_ref[...],
                  preferred_element_type=jnp.float32)
    o_ref[...] = agg / den + b_ref[...]


def _row_spec(c):
    return pl.BlockSpec((_BN, c), lambda i: (i, 0))


def _full_spec(shape):
    return pl.BlockSpec(shape, lambda i: tuple(0 for _ in shape))


def _tc0(x, w, av, bv):
    cin, cout = w.shape
    return pl.pallas_call(
        _tc0_body,
        grid=(NP // _BN,),
        in_specs=[_row_spec(cin), _full_spec(w.shape), _full_spec(av.shape),
                  _full_spec(bv.shape)],
        out_specs=[_row_spec(cout), _row_spec(1), _row_spec(1)],
        out_shape=[
            jax.ShapeDtypeStruct((NP, cout), jnp.float32),
            jax.ShapeDtypeStruct((NP, 1), jnp.float32),
            jax.ShapeDtypeStruct((NP, 1), jnp.float32),
        ],
    )(x, w, av, bv)


def _tc_mid(u, den, b, w, av, bv):
    cin, cout = w.shape
    return pl.pallas_call(
        _tc_mid_body,
        grid=(NP // _BN,),
        in_specs=[_row_spec(cin), _row_spec(cin), _row_spec(1), _row_spec(1),
                  _full_spec((1, cin)), _full_spec(w.shape),
                  _full_spec(av.shape), _full_spec(bv.shape)],
        out_specs=[_row_spec(cout), _row_spec(1), _row_spec(1)],
        out_shape=[
            jax.ShapeDtypeStruct((NP, cout), jnp.float32),
            jax.ShapeDtypeStruct((NP, 1), jnp.float32),
            jax.ShapeDtypeStruct((NP, 1), jnp.float32),
        ],
    )(u[0], u[1], den[0].reshape(NP, 1), den[1].reshape(NP, 1),
      b.reshape(1, cin), w, av, bv)


def _tc_pre(u, den, b, w, av, bv):
    cin = u.shape[-1]
    return pl.pallas_call(
        _tc_pre_body,
        grid=(NP // _BN,),
        in_specs=[_row_spec(cin), _row_spec(cin), _row_spec(1), _row_spec(1),
                  _full_spec((1, cin)), _full_spec(w.shape),
                  _full_spec(av.shape), _full_spec(bv.shape)],
        out_specs=[_row_spec(cin), _row_spec(1), _row_spec(1)],
        out_shape=[
            jax.ShapeDtypeStruct((NP, cin), jnp.float32),
            jax.ShapeDtypeStruct((NP, 1), jnp.float32),
            jax.ShapeDtypeStruct((NP, 1), jnp.float32),
        ],
    )(u[0], u[1], den[0].reshape(NP, 1), den[1].reshape(NP, 1),
      b.reshape(1, cin), w, av, bv)


def _tc_fin(u, den, b, w):
    cin, cout = w.shape
    return pl.pallas_call(
        _tc_fin_body,
        grid=(NP // _BN,),
        in_specs=[_row_spec(cin), _row_spec(cin), _row_spec(1), _row_spec(1),
                  _full_spec((1, cout)), _full_spec(w.shape)],
        out_specs=_row_spec(cout),
        out_shape=jax.ShapeDtypeStruct((NP, cout), jnp.float32),
    )(u[0], u[1], den[0].reshape(NP, 1), den[1].reshape(NP, 1),
      b.reshape(1, cout), w)


def kernel(x, edge_index, W0, a_src0, a_dst0, b0, W1, a_src1, a_dst1, b1,
           W2, a_src2, a_dst2, b2):
    loop = jnp.arange(N, dtype=edge_index.dtype)
    src = jnp.concatenate([edge_index[0], loop])
    dst = jnp.concatenate([edge_index[1], loop])
    pad = jnp.full((EP - E1,), N, dtype=edge_index.dtype)
    src_f = jnp.concatenate([src, pad])
    dst_f = jnp.concatenate([dst, pad])

    xp = jnp.pad(x, ((0, NP - N), (0, 0)))

    def sc_call(h, s, d):
        c = h.shape[-1]
        e3 = (NW, EP // (NW * 128), 128)
        sv, dv = s.reshape(NP), d.reshape(NP)
        if c > 64:
            su = lax.bitcast_convert_type(
                sv.astype(jnp.bfloat16), jnp.uint16).astype(jnp.uint32)
            du = lax.bitcast_convert_type(
                dv.astype(jnp.bfloat16), jnp.uint16).astype(jnp.uint32)
            aa = lax.bitcast_convert_type((su << 16) | du, jnp.int32)
            sv = dv = aa
        return _sc_edge_kernel(c)(h, sv, dv,
                                  src_f.reshape(e3), dst_f.reshape(e3))

    h, s, d = _tc0(xp, W0, a_src0.reshape(-1, 1), a_dst0.reshape(-1, 1))
    u, den = sc_call(h, s, d)
    h, s, d = _tc_mid(u, den, b0, W1, a_src1.reshape(-1, 1),
                      a_dst1.reshape(-1, 1))
    u, den = sc_call(h, s, d)
    h, s, d = _tc_pre(u, den, b1, W2, a_src2.reshape(-1, 1),
                      a_dst2.reshape(-1, 1))
    u, den = sc_call(h, s, d)
    out = _tc_fin(u, den, b2, W2)
    return out[:N]


# speedup vs baseline: 2.4817x; 1.3249x over previous
import functools

import jax
import jax.numpy as jnp
from jax import lax
from jax.experimental import pallas as pl
from jax.experimental.pallas import tpu as pltpu
from jax.experimental.pallas import tpu_sc as plsc

N = 10000
D_IN = 128
NP = 10240
NC = 2
NS = 16
NW = NC * NS
E1 = 320000 + N
EP = 331776
RPT = NP // NS


@functools.cache
def _sc_edge_kernel(c: int):
    mesh = plsc.VectorSubcoreMesh(
        core_axis_name="c", subcore_axis_name="s", num_cores=NC, num_subcores=NS
    )
    blk = 128
    bpw = EP // (NW * blk)
    packed = c > 64

    def body(h_hbm, as_hbm, ad_hbm, src_hbm, dst_hbm,
             u_out, den_out,
             src_t, dst_t, as_t, ad_t, ex_t, rows_t, zden_t,
             u_sh, den_sh, sem_g, sem_s, sem_d, sem_u, sem_e):
        cid = lax.axis_index("c")
        sid = lax.axis_index("s")
        wid = cid * NS + sid

        def idx_fetch(b, slot):
            pltpu.async_copy(src_hbm.at[wid, b], src_t.at[slot],
                             sem_s.at[slot])
            pltpu.async_copy(dst_hbm.at[wid, b], dst_t.at[slot],
                             sem_d.at[slot])

        def idx_wait(slot):
            pltpu.make_async_copy(src_hbm.at[wid, 0], src_t.at[slot],
                                  sem_s.at[slot]).wait()
            pltpu.make_async_copy(dst_hbm.at[wid, 0], dst_t.at[slot],
                                  sem_d.at[slot]).wait()

        def gather_start(b, buf):
            pltpu.async_copy(h_hbm.at[src_t.at[lax.rem(b, 6)]],
                             rows_t.at[buf], sem_g.at[buf])

        def gather_wait(buf):
            pltpu.make_async_copy(h_hbm.at[src_t.at[0]], rows_t.at[buf],
                                  sem_g.at[buf]).wait()

        def scatter_start(slot, buf):
            pltpu.async_copy(rows_t.at[buf], u_sh.at[dst_t.at[slot]],
                             sem_u.at[buf], add=True)
            pltpu.async_copy(ex_t.at[buf], den_sh.at[dst_t.at[slot]],
                             sem_e.at[buf], add=True)

        def scatter_wait(buf):
            pltpu.make_async_copy(rows_t.at[buf], u_sh.at[dst_t.at[0]],
                                  sem_u.at[buf]).wait()
            pltpu.make_async_copy(ex_t.at[buf], den_sh.at[dst_t.at[0]],
                                  sem_e.at[buf]).wait()

        idx_fetch(0, 0)
        idx_fetch(1, 1)
        idx_fetch(2, 2)
        pltpu.sync_copy(as_hbm, as_t)
        if not packed:
            pltpu.sync_copy(ad_hbm, ad_t)

        zero = jnp.zeros((16,), jnp.float32)

        @plsc.parallel_loop(0, blk)
        def _zrow(r):
            for j in range(c // 16):
                rows_t[0, r, pl.ds(j * 16, 16)] = zero

        @plsc.parallel_loop(0, RPT // 16)
        def _zden(i):
            zden_t[pl.ds(i * 16, 16)] = zero

        full, rem = divmod(RPT, blk)
        for i in range(full):
            pltpu.sync_copy(rows_t.at[0],
                            u_sh.at[pl.ds(sid * RPT + i * blk, blk)])
        if rem:
            pltpu.sync_copy(rows_t.at[0, pl.ds(0, rem)],
                            u_sh.at[pl.ds(sid * RPT + full * blk, rem)])
        pltpu.sync_copy(zden_t, den_sh.at[pl.ds(sid * RPT, RPT)])
        plsc.subcore_barrier()

        idx_wait(0)
        gather_start(0, 0)
        idx_wait(1)
        gather_start(1, 1)

        def blk_body(b, _):
            slot = lax.rem(b, 6)
            buf = lax.rem(b, 3)

            @plsc.parallel_loop(0, blk // 16, unroll=2)
            def _alpha(g):
                s = src_t[slot, pl.ds(g * 16, 16)]
                d = dst_t[slot, pl.ds(g * 16, 16)]
                if packed:
                    ws = plsc.load_gather(as_t, [s])
                    wd = plsc.load_gather(as_t, [d])
                    av = lax.bitcast_convert_type(
                        ws & jnp.int32(-65536), jnp.float32)
                    dv = lax.bitcast_convert_type(
                        lax.shift_left(wd, 16), jnp.float32)
                    al = av + dv
                else:
                    al = (plsc.load_gather(as_t, [s])
                          + plsc.load_gather(ad_t, [d]))
                al = jnp.where(al >= 0, al, al * jnp.float32(0.2))
                ex_t[buf, pl.ds(g * 16, 16)] = jnp.exp(al)

            gather_wait(buf)

            @plsc.parallel_loop(0, blk // 16, unroll=2)
            def _scale(g):
                exv = ex_t[buf, pl.ds(g * 16, 16)]
                for r in range(16):
                    es = exv[jnp.full((16,), r, jnp.int32)]
                    row = g * 16 + r
                    for j in range(c // 16):
                        rows_t[buf, row, pl.ds(j * 16, 16)] = (
                            rows_t[buf, row, pl.ds(j * 16, 16)] * es
                        )

            @pl.when(b >= 2)
            def _drain_prev():
                scatter_wait(lax.rem(b - 2, 3))

            @pl.when(b + 2 < bpw)
            def _next_gather():
                idx_wait(lax.rem(b + 2, 6))
                gather_start(b + 2, lax.rem(b + 2, 3))

            @pl.when(b + 3 < bpw)
            def _next_idx():
                idx_fetch(b + 3, lax.rem(b + 3, 6))

            scatter_start(slot, buf)
            return 0

        lax.fori_loop(0, bpw, blk_body, 0)
        scatter_wait(lax.rem(bpw - 2, 3))
        scatter_wait(lax.rem(bpw - 1, 3))
        plsc.subcore_barrier()

        pltpu.sync_copy(u_sh.at[pl.ds(sid * RPT, RPT)],
                        u_out.at[cid, pl.ds(sid * RPT, RPT)])
        pltpu.sync_copy(den_sh.at[pl.ds(sid * RPT, RPT)],
                        den_out.at[cid, pl.ds(sid * RPT, RPT)])

    return pl.kernel(
        body,
        out_type=(
            jax.ShapeDtypeStruct((NC, NP, c), jnp.float32),
            jax.ShapeDtypeStruct((NC, NP), jnp.float32),
        ),
        mesh=mesh,
        compiler_params=pltpu.CompilerParams(
            needs_layout_passes=False, use_tc_tiling_on_sc=False),
        scratch_types=[
            pltpu.VMEM((6, blk), jnp.int32),
            pltpu.VMEM((6, blk), jnp.int32),
            pltpu.VMEM((NP,), jnp.int32 if packed else jnp.float32),
            pltpu.VMEM((16,) if packed else (NP,), jnp.float32),
            pltpu.VMEM((3, blk), jnp.float32),
            pltpu.VMEM((3, blk, c), jnp.float32),
            pltpu.VMEM((RPT,), jnp.float32),
            pltpu.VMEM_SHARED((NP, c), jnp.float32),
            pltpu.VMEM_SHARED((NP,), jnp.float32),
            pltpu.SemaphoreType.DMA((3,)),
            pltpu.SemaphoreType.DMA((6,)),
            pltpu.SemaphoreType.DMA((6,)),
            pltpu.SemaphoreType.DMA((3,)),
            pltpu.SemaphoreType.DMA((3,)),
        ],
    )


_BN = 1024


def _tc0_body(x_ref, w_ref, av_ref, bv_ref, h_ref, s_ref, d_ref):
    h = jnp.dot(x_ref[...], w_ref[...], preferred_element_type=jnp.float32)
    h_ref[...] = h
    s_ref[...] = jnp.dot(h, av_ref[...], preferred_element_type=jnp.float32)
    d_ref[...] = jnp.dot(h, bv_ref[...], preferred_element_type=jnp.float32)


def _tc_mid_body(u0_ref, u1_ref, n0_ref, n1_ref, b_ref, w_ref, av_ref, bv_ref,
                 h_ref, s_ref, d_ref):
    den = n0_ref[...] + n1_ref[...] + jnp.float32(1e-16)
    xact = jnp.maximum((u0_ref[...] + u1_ref[...]) / den + b_ref[...], 0.0)
    h = jnp.dot(xact, w_ref[...], preferred_element_type=jnp.float32)
    h_ref[...] = h
    s_ref[...] = jnp.dot(h, av_ref[...], preferred_element_type=jnp.float32)
    d_ref[...] = jnp.dot(h, bv_ref[...], preferred_element_type=jnp.float32)


def _tc_pre_body(u0_ref, u1_ref, n0_ref, n1_ref, b_ref, w_ref, av_ref, bv_ref,
                 h_ref, s_ref, d_ref):
    den = n0_ref[...] + n1_ref[...] + jnp.float32(1e-16)
    xact = jnp.maximum((u0_ref[...] + u1_ref[...]) / den + b_ref[...], 0.0)
    h_ref[...] = xact
    wa = jnp.dot(w_ref[...], av_ref[...], preferred_element_type=jnp.float32)
    wb = jnp.dot(w_ref[...], bv_ref[...], preferred_element_type=jnp.float32)
    s_ref[...] = jnp.dot(xact, wa, preferred_element_type=jnp.float32)
    d_ref[...] = jnp.dot(xact, wb, preferred_element_type=jnp.float32)


def _tc_fin_body(u0_ref, u1_ref, n0_ref, n1_ref, b_ref, w_ref, o_ref):
    den = n0_ref[...] + n1_ref[...] + jnp.float32(1e-16)
    agg = jnp.dot(u0_ref[...] + u1_ref[...], w_ref[...],
                  preferred_element_type=jnp.float32)
    o_ref[...] = agg / den + b_ref[...]


def _row_spec(c):
    return pl.BlockSpec((_BN, c), lambda i: (i, 0))


def _full_spec(shape):
    return pl.BlockSpec(shape, lambda i: tuple(0 for _ in shape))


def _tc0(x, w, av, bv):
    cin, cout = w.shape
    return pl.pallas_call(
        _tc0_body,
        grid=(NP // _BN,),
        in_specs=[_row_spec(cin), _full_spec(w.shape), _full_spec(av.shape),
                  _full_spec(bv.shape)],
        out_specs=[_row_spec(cout), _row_spec(1), _row_spec(1)],
        out_shape=[
            jax.ShapeDtypeStruct((NP, cout), jnp.float32),
            jax.ShapeDtypeStruct((NP, 1), jnp.float32),
            jax.ShapeDtypeStruct((NP, 1), jnp.float32),
        ],
    )(x, w, av, bv)


def _tc_mid(u, den, b, w, av, bv):
    cin, cout = w.shape
    return pl.pallas_call(
        _tc_mid_body,
        grid=(NP // _BN,),
        in_specs=[_row_spec(cin), _row_spec(cin), _row_spec(1), _row_spec(1),
                  _full_spec((1, cin)), _full_spec(w.shape),
                  _full_spec(av.shape), _full_spec(bv.shape)],
        out_specs=[_row_spec(cout), _row_spec(1), _row_spec(1)],
        out_shape=[
            jax.ShapeDtypeStruct((NP, cout), jnp.float32),
            jax.ShapeDtypeStruct((NP, 1), jnp.float32),
            jax.ShapeDtypeStruct((NP, 1), jnp.float32),
        ],
    )(u[0], u[1], den[0].reshape(NP, 1), den[1].reshape(NP, 1),
      b.reshape(1, cin), w, av, bv)


def _tc_pre(u, den, b, w, av, bv):
    cin = u.shape[-1]
    return pl.pallas_call(
        _tc_pre_body,
        grid=(NP // _BN,),
        in_specs=[_row_spec(cin), _row_spec(cin), _row_spec(1), _row_spec(1),
                  _full_spec((1, cin)), _full_spec(w.shape),
                  _full_spec(av.shape), _full_spec(bv.shape)],
        out_specs=[_row_spec(cin), _row_spec(1), _row_spec(1)],
        out_shape=[
            jax.ShapeDtypeStruct((NP, cin), jnp.float32),
            jax.ShapeDtypeStruct((NP, 1), jnp.float32),
            jax.ShapeDtypeStruct((NP, 1), jnp.float32),
        ],
    )(u[0], u[1], den[0].reshape(NP, 1), den[1].reshape(NP, 1),
      b.reshape(1, cin), w, av, bv)


def _tc_fin(u, den, b, w):
    cin, cout = w.shape
    return pl.pallas_call(
        _tc_fin_body,
        grid=(NP // _BN,),
        in_specs=[_row_spec(cin), _row_spec(cin), _row_spec(1), _row_spec(1),
                  _full_spec((1, cout)), _full_spec(w.shape)],
        out_specs=_row_spec(cout),
        out_shape=jax.ShapeDtypeStruct((NP, cout), jnp.float32),
    )(u[0], u[1], den[0].reshape(NP, 1), den[1].reshape(NP, 1),
      b.reshape(1, cout), w)


def kernel(x, edge_index, W0, a_src0, a_dst0, b0, W1, a_src1, a_dst1, b1,
           W2, a_src2, a_dst2, b2):
    loop = jnp.arange(N, dtype=edge_index.dtype)
    src = jnp.concatenate([edge_index[0], loop])
    dst = jnp.concatenate([edge_index[1], loop])
    pad = jnp.full((EP - E1,), N, dtype=edge_index.dtype)
    src_f = jnp.concatenate([src, pad])
    dst_f = jnp.concatenate([dst, pad])

    xp = jnp.pad(x, ((0, NP - N), (0, 0)))

    def sc_call(h, s, d):
        c = h.shape[-1]
        e3 = (NW, EP // (NW * 128), 128)
        sv, dv = s.reshape(NP), d.reshape(NP)
        if c > 64:
            su = lax.bitcast_convert_type(
                sv.astype(jnp.bfloat16), jnp.uint16).astype(jnp.uint32)
            du = lax.bitcast_convert_type(
                dv.astype(jnp.bfloat16), jnp.uint16).astype(jnp.uint32)
            aa = lax.bitcast_convert_type((su << 16) | du, jnp.int32)
            sv = dv = aa
        return _sc_edge_kernel(c)(h, sv, dv,
                                  src_f.reshape(e3), dst_f.reshape(e3))

    h, s, d = _tc0(xp, W0, a_src0.reshape(-1, 1), a_dst0.reshape(-1, 1))
    u, den = sc_call(h, s, d)
    h, s, d = _tc_mid(u, den, b0, W1, a_src1.reshape(-1, 1),
                      a_dst1.reshape(-1, 1))
    u, den = sc_call(h, s, d)
    h, s, d = _tc_pre(u, den, b1, W2, a_src2.reshape(-1, 1),
                      a_dst2.reshape(-1, 1))
    u, den = sc_call(h, s, d)
    out = _tc_fin(u, den, b2, W2)
    return out[:N]


# speedup vs baseline: 2.7144x; 1.0938x over previous
import functools

import jax
import jax.numpy as jnp
from jax import lax
from jax.experimental import pallas as pl
from jax.experimental.pallas import tpu as pltpu
from jax.experimental.pallas import tpu_sc as plsc

N = 10000
D_IN = 128
NP = 10240
NC = 2
NS = 16
NW = NC * NS
E1 = 320000 + N
EP = 331776
RPT = NP // NS


@functools.cache
def _sc_edge_kernel(c: int):
    mesh = plsc.VectorSubcoreMesh(
        core_axis_name="c", subcore_axis_name="s", num_cores=NC, num_subcores=NS
    )
    blk = 128
    bpw = EP // (NW * blk)
    packed = c > 64

    def body(h_hbm, as_hbm, ad_hbm, edge_hbm,
             u_out, den_out,
             src_t, dst_t, as_t, ad_t, ex_t, rows_t, zden_t,
             u_sh, den_sh, sem_g, sem_s, sem_d, sem_u, sem_e):
        cid = lax.axis_index("c")
        sid = lax.axis_index("s")
        wid = cid * NS + sid

        def idx_fetch(b, slot):
            pltpu.async_copy(edge_hbm.at[0, wid, b], src_t.at[slot],
                             sem_s.at[slot])
            pltpu.async_copy(edge_hbm.at[1, wid, b], dst_t.at[slot],
                             sem_d.at[slot])

        def idx_wait(slot):
            pltpu.make_async_copy(edge_hbm.at[0, wid, 0], src_t.at[slot],
                                  sem_s.at[slot]).wait()
            pltpu.make_async_copy(edge_hbm.at[1, wid, 0], dst_t.at[slot],
                                  sem_d.at[slot]).wait()

        def gather_start(b, buf):
            pltpu.async_copy(h_hbm.at[src_t.at[lax.rem(b, 6)]],
                             rows_t.at[buf], sem_g.at[buf])

        def gather_wait(buf):
            pltpu.make_async_copy(h_hbm.at[src_t.at[0]], rows_t.at[buf],
                                  sem_g.at[buf]).wait()

        def scatter_start(slot, buf):
            pltpu.async_copy(rows_t.at[buf], u_sh.at[dst_t.at[slot]],
                             sem_u.at[buf], add=True)
            pltpu.async_copy(ex_t.at[buf], den_sh.at[dst_t.at[slot]],
                             sem_e.at[buf], add=True)

        def scatter_wait(buf):
            pltpu.make_async_copy(rows_t.at[buf], u_sh.at[dst_t.at[0]],
                                  sem_u.at[buf]).wait()
            pltpu.make_async_copy(ex_t.at[buf], den_sh.at[dst_t.at[0]],
                                  sem_e.at[buf]).wait()

        idx_fetch(0, 0)
        idx_fetch(1, 1)
        idx_fetch(2, 2)
        pltpu.sync_copy(as_hbm, as_t)
        if not packed:
            pltpu.sync_copy(ad_hbm, ad_t)

        zero = jnp.zeros((16,), jnp.float32)

        @plsc.parallel_loop(0, blk)
        def _zrow(r):
            for j in range(c // 16):
                rows_t[0, r, pl.ds(j * 16, 16)] = zero

        @plsc.parallel_loop(0, RPT // 16)
        def _zden(i):
            zden_t[pl.ds(i * 16, 16)] = zero

        full, rem = divmod(RPT, blk)
        for i in range(full):
            pltpu.sync_copy(rows_t.at[0],
                            u_sh.at[pl.ds(sid * RPT + i * blk, blk)])
        if rem:
            pltpu.sync_copy(rows_t.at[0, pl.ds(0, rem)],
                            u_sh.at[pl.ds(sid * RPT + full * blk, rem)])
        pltpu.sync_copy(zden_t, den_sh.at[pl.ds(sid * RPT, RPT)])
        plsc.subcore_barrier()

        idx_wait(0)
        gather_start(0, 0)
        idx_wait(1)
        gather_start(1, 1)

        def blk_body(b, _):
            slot = lax.rem(b, 6)
            buf = lax.rem(b, 3)

            @plsc.parallel_loop(0, blk // 16, unroll=2)
            def _alpha(g):
                s = src_t[slot, pl.ds(g * 16, 16)]
                d = dst_t[slot, pl.ds(g * 16, 16)]
                if packed:
                    ws = plsc.load_gather(as_t, [s])
                    wd = plsc.load_gather(as_t, [d])
                    av = lax.bitcast_convert_type(
                        ws & jnp.int32(-65536), jnp.float32)
                    dv = lax.bitcast_convert_type(
                        lax.shift_left(wd, 16), jnp.float32)
                    al = av + dv
                else:
                    al = (plsc.load_gather(as_t, [s])
                          + plsc.load_gather(ad_t, [d]))
                al = jnp.where(al >= 0, al, al * jnp.float32(0.2))
                ex_t[buf, pl.ds(g * 16, 16)] = jnp.exp(al)

            gather_wait(buf)

            @plsc.parallel_loop(0, blk // 16, unroll=2)
            def _scale(g):
                exv = ex_t[buf, pl.ds(g * 16, 16)]
                for r in range(16):
                    es = exv[jnp.full((16,), r, jnp.int32)]
                    row = g * 16 + r
                    for j in range(c // 16):
                        rows_t[buf, row, pl.ds(j * 16, 16)] = (
                            rows_t[buf, row, pl.ds(j * 16, 16)] * es
                        )

            @pl.when(b >= 2)
            def _drain_prev():
                scatter_wait(lax.rem(b - 2, 3))

            @pl.when(b + 2 < bpw)
            def _next_gather():
                idx_wait(lax.rem(b + 2, 6))
                gather_start(b + 2, lax.rem(b + 2, 3))

            @pl.when(b + 3 < bpw)
            def _next_idx():
                idx_fetch(b + 3, lax.rem(b + 3, 6))

            scatter_start(slot, buf)
            return 0

        lax.fori_loop(0, bpw, blk_body, 0)
        scatter_wait(lax.rem(bpw - 2, 3))
        scatter_wait(lax.rem(bpw - 1, 3))
        plsc.subcore_barrier()

        pltpu.sync_copy(u_sh.at[pl.ds(sid * RPT, RPT)],
                        u_out.at[cid, pl.ds(sid * RPT, RPT)])
        pltpu.sync_copy(den_sh.at[pl.ds(sid * RPT, RPT)],
                        den_out.at[cid, pl.ds(sid * RPT, RPT)])

    return pl.kernel(
        body,
        out_type=(
            jax.ShapeDtypeStruct((NC, NP, c), jnp.float32),
            jax.ShapeDtypeStruct((NC, NP), jnp.float32),
        ),
        mesh=mesh,
        compiler_params=pltpu.CompilerParams(
            needs_layout_passes=False, use_tc_tiling_on_sc=False),
        scratch_types=[
            pltpu.VMEM((6, blk), jnp.int32),
            pltpu.VMEM((6, blk), jnp.int32),
            pltpu.VMEM((NP,), jnp.int32 if packed else jnp.float32),
            pltpu.VMEM((16,) if packed else (NP,), jnp.float32),
            pltpu.VMEM((3, blk), jnp.float32),
            pltpu.VMEM((3, blk, c), jnp.float32),
            pltpu.VMEM((RPT,), jnp.float32),
            pltpu.VMEM_SHARED((NP, c), jnp.float32),
            pltpu.VMEM_SHARED((NP,), jnp.float32),
            pltpu.SemaphoreType.DMA((3,)),
            pltpu.SemaphoreType.DMA((6,)),
            pltpu.SemaphoreType.DMA((6,)),
            pltpu.SemaphoreType.DMA((3,)),
            pltpu.SemaphoreType.DMA((3,)),
        ],
    )


_BN = 1024


def _tc0_body(x_ref, w_ref, av_ref, bv_ref, h_ref, s_ref, d_ref):
    h = jnp.dot(x_ref[...], w_ref[...], preferred_element_type=jnp.float32)
    h_ref[...] = h
    s_ref[...] = jnp.dot(h, av_ref[...], preferred_element_type=jnp.float32)
    d_ref[...] = jnp.dot(h, bv_ref[...], preferred_element_type=jnp.float32)


def _tc_mid_body(u_ref, n_ref, b_ref, w_ref, av_ref, bv_ref,
                 h_ref, s_ref, d_ref):
    den = n_ref[0] + n_ref[1] + jnp.float32(1e-16)
    xact = jnp.maximum((u_ref[0] + u_ref[1]) / den + b_ref[...], 0.0)
    h = jnp.dot(xact, w_ref[...], preferred_element_type=jnp.float32)
    h_ref[...] = h
    s_ref[...] = jnp.dot(h, av_ref[...], preferred_element_type=jnp.float32)
    d_ref[...] = jnp.dot(h, bv_ref[...], preferred_element_type=jnp.float32)


def _tc_pre_body(u_ref, n_ref, b_ref, w_ref, av_ref, bv_ref,
                 h_ref, s_ref, d_ref):
    den = n_ref[0] + n_ref[1] + jnp.float32(1e-16)
    xact = jnp.maximum((u_ref[0] + u_ref[1]) / den + b_ref[...], 0.0)
    h_ref[...] = xact
    wa = jnp.dot(w_ref[...], av_ref[...], preferred_element_type=jnp.float32)
    wb = jnp.dot(w_ref[...], bv_ref[...], preferred_element_type=jnp.float32)
    s_ref[...] = jnp.dot(xact, wa, preferred_element_type=jnp.float32)
    d_ref[...] = jnp.dot(xact, wb, preferred_element_type=jnp.float32)


def _tc_fin_body(u_ref, n_ref, b_ref, w_ref, o_ref):
    den = n_ref[0] + n_ref[1] + jnp.float32(1e-16)
    agg = jnp.dot(u_ref[0] + u_ref[1], w_ref[...],
                  preferred_element_type=jnp.float32)
    o_ref[...] = agg / den + b_ref[...]


def _row_spec(c, bn=_BN):
    return pl.BlockSpec((bn, c), lambda i: (i, 0))


def _u_spec(c, bn=_BN):
    return pl.BlockSpec((NC, bn, c), lambda i: (0, i, 0))


def _n_spec(bn=_BN):
    return pl.BlockSpec((NC, bn, 1), lambda i: (0, i, 0))


def _full_spec(shape):
    return pl.BlockSpec(shape, lambda i: tuple(0 for _ in shape))


def _tc0(x, w, av, bv):
    cin, cout = w.shape
    return pl.pallas_call(
        _tc0_body,
        grid=(NP // _BN,),
        in_specs=[_row_spec(cin), _full_spec(w.shape), _full_spec(av.shape),
                  _full_spec(bv.shape)],
        out_specs=[_row_spec(cout), _row_spec(1), _row_spec(1)],
        out_shape=[
            jax.ShapeDtypeStruct((NP, cout), jnp.float32),
            jax.ShapeDtypeStruct((NP, 1), jnp.float32),
            jax.ShapeDtypeStruct((NP, 1), jnp.float32),
        ],
    )(x, w, av, bv)


def _tc_layer(body, u, den, b, w, av, bv, cout):
    cin = u.shape[-1]
    return pl.pallas_call(
        body,
        grid=(NP // _BN,),
        in_specs=[_u_spec(cin), _n_spec(),
                  _full_spec((1, cin)), _full_spec(w.shape),
                  _full_spec(av.shape), _full_spec(bv.shape)],
        out_specs=[_row_spec(cout), _row_spec(1), _row_spec(1)],
        out_shape=[
            jax.ShapeDtypeStruct((NP, cout), jnp.float32),
            jax.ShapeDtypeStruct((NP, 1), jnp.float32),
            jax.ShapeDtypeStruct((NP, 1), jnp.float32),
        ],
    )(u, den.reshape(NC, NP, 1), b.reshape(1, cin), w, av, bv)


def _tc_fin(u, den, b, w):
    cin, cout = w.shape
    bn = 1000
    return pl.pallas_call(
        _tc_fin_body,
        grid=(N // bn,),
        in_specs=[_u_spec(cin, bn), _n_spec(bn),
                  _full_spec((1, cout)), _full_spec(w.shape)],
        out_specs=_row_spec(cout, bn),
        out_shape=jax.ShapeDtypeStruct((N, cout), jnp.float32),
    )(u, den.reshape(NC, NP, 1), b.reshape(1, cout), w)


def kernel(x, edge_index, W0, a_src0, a_dst0, b0, W1, a_src1, a_dst1, b1,
           W2, a_src2, a_dst2, b2):
    loop2 = jnp.broadcast_to(jnp.arange(N, dtype=edge_index.dtype), (2, N))
    pad2 = jnp.full((2, EP - E1), N, dtype=edge_index.dtype)
    edges = jnp.concatenate([edge_index, loop2, pad2], axis=1).reshape(
        2, NW, EP // (NW * 128), 128)

    def sc_call(h, s, d):
        return _sc_edge_kernel(h.shape[-1])(
            h, s.reshape(NP), d.reshape(NP), edges)

    h, s, d = _tc0(x, W0, a_src0.reshape(-1, 1), a_dst0.reshape(-1, 1))
    u, den = sc_call(h, s, d)
    h, s, d = _tc_layer(_tc_mid_body, u, den, b0, W1, a_src1.reshape(-1, 1),
                        a_dst1.reshape(-1, 1), 64)
    u, den = sc_call(h, s, d)
    h, s, d = _tc_layer(_tc_pre_body, u, den, b1, W2, a_src2.reshape(-1, 1),
                        a_dst2.reshape(-1, 1), 64)
    u, den = sc_call(h, s, d)
    return _tc_fin(u, den, b2, W2)


# speedup vs baseline: 3.0496x; 1.1235x over previous
import functools

import jax
import jax.numpy as jnp
from jax import lax
from jax.experimental import pallas as pl
from jax.experimental.pallas import tpu as pltpu
from jax.experimental.pallas import tpu_sc as plsc

N = 10000
D_IN = 128
NP = 10240
NC = 2
NS = 16
NW = NC * NS
E1 = 320000 + N
EP = 331776
RPT = NP // NS


@functools.cache
def _sc_edge_kernel(c: int):
    mesh = plsc.VectorSubcoreMesh(
        core_axis_name="c", subcore_axis_name="s", num_cores=NC, num_subcores=NS
    )
    blk = 128
    bpw = EP // (NW * blk)
    packed = c > 64

    def body(h_hbm, as_hbm, ad_hbm, edge_hbm,
             u_out, den_out,
             src_t, dst_t, as_t, ad_t, ex_t, rows_t, zden_t,
             u_sh, den_sh, sem_g, sem_s, sem_d, sem_u, sem_e):
        cid = lax.axis_index("c")
        sid = lax.axis_index("s")
        wid = cid * NS + sid

        def idx_fetch(b, slot):
            pltpu.async_copy(edge_hbm.at[0, wid, b], src_t.at[slot],
                             sem_s.at[slot])
            pltpu.async_copy(edge_hbm.at[1, wid, b], dst_t.at[slot],
                             sem_d.at[slot])

        def idx_wait(slot):
            pltpu.make_async_copy(edge_hbm.at[0, wid, 0], src_t.at[slot],
                                  sem_s.at[slot]).wait()
            pltpu.make_async_copy(edge_hbm.at[1, wid, 0], dst_t.at[slot],
                                  sem_d.at[slot]).wait()

        def gather_start(b, buf):
            pltpu.async_copy(h_hbm.at[src_t.at[lax.rem(b, 6)]],
                             rows_t.at[buf], sem_g.at[buf])

        def gather_wait(buf):
            pltpu.make_async_copy(h_hbm.at[src_t.at[0]], rows_t.at[buf],
                                  sem_g.at[buf]).wait()

        def scatter_start(slot, buf):
            pltpu.async_copy(rows_t.at[buf], u_sh.at[dst_t.at[slot]],
                             sem_u.at[buf], add=True)
            pltpu.async_copy(ex_t.at[buf], den_sh.at[dst_t.at[slot]],
                             sem_e.at[buf], add=True)

        def scatter_wait(buf):
            pltpu.make_async_copy(rows_t.at[buf], u_sh.at[dst_t.at[0]],
                                  sem_u.at[buf]).wait()
            pltpu.make_async_copy(ex_t.at[buf], den_sh.at[dst_t.at[0]],
                                  sem_e.at[buf]).wait()

        idx_fetch(0, 0)
        idx_fetch(1, 1)
        idx_fetch(2, 2)
        pltpu.sync_copy(as_hbm, as_t)
        if not packed:
            pltpu.sync_copy(ad_hbm, ad_t)

        zero = jnp.zeros((16,), jnp.float32)

        @plsc.parallel_loop(0, blk)
        def _zrow(r):
            for j in range(c // 16):
                rows_t[0, r, pl.ds(j * 16, 16)] = zero

        @plsc.parallel_loop(0, RPT // 16)
        def _zden(i):
            zden_t[pl.ds(i * 16, 16)] = zero

        full, rem = divmod(RPT, blk)
        for i in range(full):
            pltpu.sync_copy(rows_t.at[0],
                            u_sh.at[pl.ds(sid * RPT + i * blk, blk)])
        if rem:
            pltpu.sync_copy(rows_t.at[0, pl.ds(0, rem)],
                            u_sh.at[pl.ds(sid * RPT + full * blk, rem)])
        pltpu.sync_copy(zden_t, den_sh.at[pl.ds(sid * RPT, RPT)])
        plsc.subcore_barrier()

        idx_wait(0)
        gather_start(0, 0)
        idx_wait(1)
        gather_start(1, 1)

        def blk_body(b, _):
            slot = lax.rem(b, 6)
            buf = lax.rem(b, 3)

            @plsc.parallel_loop(0, blk // 16, unroll=2)
            def _alpha(g):
                s = src_t[slot, pl.ds(g * 16, 16)]
                d = dst_t[slot, pl.ds(g * 16, 16)]
                if packed:
                    ws = plsc.load_gather(as_t, [s])
                    wd = plsc.load_gather(as_t, [d])
                    av = lax.bitcast_convert_type(
                        ws & jnp.int32(-65536), jnp.float32)
                    dv = lax.bitcast_convert_type(
                        lax.shift_left(wd, 16), jnp.float32)
                    al = av + dv
                else:
                    al = (plsc.load_gather(as_t, [s])
                          + plsc.load_gather(ad_t, [d]))
                al = jnp.where(al >= 0, al, al * jnp.float32(0.2))
                ex_t[buf, pl.ds(g * 16, 16)] = jnp.exp(al)

            gather_wait(buf)

            @plsc.parallel_loop(0, blk // 16, unroll=2)
            def _scale(g):
                exv = ex_t[buf, pl.ds(g * 16, 16)]
                for r in range(16):
                    es = exv[jnp.full((16,), r, jnp.int32)]
                    row = g * 16 + r
                    for j in range(c // 16):
                        rows_t[buf, row, pl.ds(j * 16, 16)] = (
                            rows_t[buf, row, pl.ds(j * 16, 16)] * es
                        )

            @pl.when(b >= 2)
            def _drain_prev():
                scatter_wait(lax.rem(b - 2, 3))

            @pl.when(b + 2 < bpw)
            def _next_gather():
                idx_wait(lax.rem(b + 2, 6))
                gather_start(b + 2, lax.rem(b + 2, 3))

            @pl.when(b + 3 < bpw)
            def _next_idx():
                idx_fetch(b + 3, lax.rem(b + 3, 6))

            scatter_start(slot, buf)
            return 0

        lax.fori_loop(0, bpw, blk_body, 0)
        scatter_wait(lax.rem(bpw - 2, 3))
        scatter_wait(lax.rem(bpw - 1, 3))
        plsc.subcore_barrier()

        pltpu.sync_copy(u_sh.at[pl.ds(sid * RPT, RPT)],
                        u_out.at[cid, pl.ds(sid * RPT, RPT)])
        pltpu.sync_copy(den_sh.at[pl.ds(sid * RPT, RPT)],
                        den_out.at[cid, pl.ds(sid * RPT, RPT)])

    return pl.kernel(
        body,
        out_type=(
            jax.ShapeDtypeStruct((NC, NP, c), jnp.float32),
            jax.ShapeDtypeStruct((NC, NP), jnp.float32),
        ),
        mesh=mesh,
        compiler_params=pltpu.CompilerParams(
            needs_layout_passes=False, use_tc_tiling_on_sc=False),
        scratch_types=[
            pltpu.VMEM((6, blk), jnp.int32),
            pltpu.VMEM((6, blk), jnp.int32),
            pltpu.VMEM((NP,), jnp.int32 if packed else jnp.float32),
            pltpu.VMEM((16,) if packed else (NP,), jnp.float32),
            pltpu.VMEM((3, blk), jnp.float32),
            pltpu.VMEM((3, blk, c), jnp.float32),
            pltpu.VMEM((RPT,), jnp.float32),
            pltpu.VMEM_SHARED((NP, c), jnp.float32),
            pltpu.VMEM_SHARED((NP,), jnp.float32),
            pltpu.SemaphoreType.DMA((3,)),
            pltpu.SemaphoreType.DMA((6,)),
            pltpu.SemaphoreType.DMA((6,)),
            pltpu.SemaphoreType.DMA((3,)),
            pltpu.SemaphoreType.DMA((3,)),
        ],
    )


_BN = 1024


def _tc0_body(x_ref, w_ref, av_ref, bv_ref, h_ref, s_ref, d_ref):
    h = jnp.dot(x_ref[...], w_ref[...], preferred_element_type=jnp.float32)
    h_ref[...] = h
    s_ref[...] = jnp.dot(h, av_ref[...], preferred_element_type=jnp.float32)[:, 0]
    d_ref[...] = jnp.dot(h, bv_ref[...], preferred_element_type=jnp.float32)[:, 0]


def _tc_mid_body(u_ref, n_ref, b_ref, w_ref, av_ref, bv_ref,
                 h_ref, s_ref, d_ref):
    den = (n_ref[0, :] + n_ref[1, :])[:, None] + jnp.float32(1e-16)
    xact = jnp.maximum((u_ref[0] + u_ref[1]) / den + b_ref[...], 0.0)
    h = jnp.dot(xact, w_ref[...], preferred_element_type=jnp.float32)
    h_ref[...] = h
    s_ref[...] = jnp.dot(h, av_ref[...], preferred_element_type=jnp.float32)[:, 0]
    d_ref[...] = jnp.dot(h, bv_ref[...], preferred_element_type=jnp.float32)[:, 0]


def _tc_pre_body(u_ref, n_ref, b_ref, w_ref, av_ref, bv_ref,
                 h_ref, s_ref, d_ref):
    den = (n_ref[0, :] + n_ref[1, :])[:, None] + jnp.float32(1e-16)
    xact = jnp.maximum((u_ref[0] + u_ref[1]) / den + b_ref[...], 0.0)
    h_ref[...] = xact
    wa = jnp.dot(w_ref[...], av_ref[...], preferred_element_type=jnp.float32)
    wb = jnp.dot(w_ref[...], bv_ref[...], preferred_element_type=jnp.float32)
    s_ref[...] = jnp.dot(xact, wa, preferred_element_type=jnp.float32)[:, 0]
    d_ref[...] = jnp.dot(xact, wb, preferred_element_type=jnp.float32)[:, 0]


def _tc_fin_body(u_ref, n_ref, b_ref, w_ref, o_ref):
    den = (n_ref[0, :] + n_ref[1, :])[:, None] + jnp.float32(1e-16)
    agg = jnp.dot(u_ref[0] + u_ref[1], w_ref[...],
                  preferred_element_type=jnp.float32)
    o_ref[...] = agg / den + b_ref[...]


def _row_spec(c, bn=_BN):
    return pl.BlockSpec((bn, c), lambda i: (i, 0))


def _u_spec(c, bn=_BN):
    return pl.BlockSpec((NC, bn, c), lambda i: (0, i, 0))


def _n_spec(bn=_BN):
    return pl.BlockSpec((NC, bn), lambda i: (0, i))


def _s_spec(bn=_BN):
    return pl.BlockSpec((bn,), lambda i: (i,))


def _full_spec(shape):
    return pl.BlockSpec(shape, lambda i: tuple(0 for _ in shape))


def _tc0(x, w, av, bv):
    cin, cout = w.shape
    return pl.pallas_call(
        _tc0_body,
        grid=(NP // _BN,),
        in_specs=[_row_spec(cin), _full_spec(w.shape), _full_spec(av.shape),
                  _full_spec(bv.shape)],
        out_specs=[_row_spec(cout), _s_spec(), _s_spec()],
        out_shape=[
            jax.ShapeDtypeStruct((NP, cout), jnp.float32),
            jax.ShapeDtypeStruct((NP,), jnp.float32),
            jax.ShapeDtypeStruct((NP,), jnp.float32),
        ],
    )(x, w, av, bv)


def _tc_layer(body, u, den, b, w, av, bv, cout):
    cin = u.shape[-1]
    return pl.pallas_call(
        body,
        grid=(NP // _BN,),
        in_specs=[_u_spec(cin), _n_spec(),
                  _full_spec((1, cin)), _full_spec(w.shape),
                  _full_spec(av.shape), _full_spec(bv.shape)],
        out_specs=[_row_spec(cout), _s_spec(), _s_spec()],
        out_shape=[
            jax.ShapeDtypeStruct((NP, cout), jnp.float32),
            jax.ShapeDtypeStruct((NP,), jnp.float32),
            jax.ShapeDtypeStruct((NP,), jnp.float32),
        ],
    )(u, den, b.reshape(1, cin), w, av, bv)


def _tc_fin(u, den, b, w):
    cin, cout = w.shape
    return pl.pallas_call(
        _tc_fin_body,
        grid=(NP // _BN,),
        in_specs=[_u_spec(cin), _n_spec(),
                  _full_spec((1, cout)), _full_spec(w.shape)],
        out_specs=_row_spec(cout),
        out_shape=jax.ShapeDtypeStruct((N, cout), jnp.float32),
    )(u, den, b.reshape(1, cout), w)


def kernel(x, edge_index, W0, a_src0, a_dst0, b0, W1, a_src1, a_dst1, b1,
           W2, a_src2, a_dst2, b2):
    loop2 = jnp.broadcast_to(jnp.arange(N, dtype=edge_index.dtype), (2, N))
    pad2 = jnp.full((2, EP - E1), N, dtype=edge_index.dtype)
    edges = jnp.concatenate([edge_index, loop2, pad2], axis=1).reshape(
        2, NW, EP // (NW * 128), 128)

    def sc_call(h, s, d):
        return _sc_edge_kernel(h.shape[-1])(h, s, d, edges)

    h, s, d = _tc0(x, W0, a_src0.reshape(-1, 1), a_dst0.reshape(-1, 1))
    u, den = sc_call(h, s, d)
    h, s, d = _tc_layer(_tc_mid_body, u, den, b0, W1, a_src1.reshape(-1, 1),
                        a_dst1.reshape(-1, 1), 64)
    u, den = sc_call(h, s, d)
    h, s, d = _tc_layer(_tc_pre_body, u, den, b1, W2, a_src2.reshape(-1, 1),
                        a_dst2.reshape(-1, 1), 64)
    u, den = sc_call(h, s, d)
    return _tc_fin(u, den, b2, W2)
